# Initial kernel scaffold; baseline (speedup 1.0000x reference)
import jax
import jax.numpy as jnp
from jax.experimental import pallas as pl


def kernel(x, edge_index, pos_edge_index, neg_edge_index, W1, W2, Wlin):
    raise NotImplementedError("write your pallas kernel here")


# speedup vs baseline: 8.0406x; 8.0406x over previous
import functools

import jax
import jax.numpy as jnp
from jax import lax
from jax.experimental import pallas as pl
from jax.experimental.pallas import tpu as pltpu
from jax.experimental.pallas import tpu_sc as plsc

N = 10000
NP = 10240
D = 128
E = 320000
EP2 = 320000
NC = 2
NS = 16
NW = NC * NS
CH = 80
EPT = E // NW
NCH = EPT // CH
RPT = NP // NS
ZR = 128

_mesh = plsc.VectorSubcoreMesh(core_axis_name="c", subcore_axis_name="s")
_sc_params = pltpu.CompilerParams(use_tc_tiling_on_sc=False)


def _sc_deg(dst):
    @functools.partial(
        pl.kernel,
        out_type=jax.ShapeDtypeStruct((NC, NP, 16), jnp.float32),
        mesh=_mesh,
        scratch_types=[
            pltpu.VMEM((ZR, 16), jnp.float32),
            pltpu.VMEM((CH, 16), jnp.float32),
            pltpu.VMEM((CH,), jnp.int32),
            pltpu.VMEM_SHARED((NP, 16), jnp.float32),
            pltpu.SemaphoreType.DMA,
        ],
        compiler_params=_sc_params,
    )
    def k(dst_hbm, out_hbm, zbuf, ones_v, ib, acc, sem):
        c = lax.axis_index("c")
        s = lax.axis_index("s")
        wid = c * NS + s
        zero16 = jnp.zeros((16,), jnp.float32)
        one16 = jnp.ones((16,), jnp.float32)

        @pl.loop(0, ZR)
        def _(i):
            zbuf[i, :] = zero16

        @pl.loop(0, CH)
        def _(i):
            ones_v[i, :] = one16

        for kk in range(RPT // ZR):
            pltpu.sync_copy(zbuf, acc.at[pl.ds(s * RPT + kk * ZR, ZR)])
        plsc.subcore_barrier()

        @pl.loop(0, NCH)
        def _(j):
            base = wid * EPT + j * CH
            pltpu.sync_copy(dst_hbm.at[pl.ds(base, CH)], ib)
            pltpu.sync_copy(ones_v, acc.at[ib], add=True)

        plsc.subcore_barrier()
        for kk in range(RPT // ZR):
            r0 = s * RPT + kk * ZR
            pltpu.sync_copy(acc.at[pl.ds(r0, ZR)], out_hbm.at[c].at[pl.ds(r0, ZR)])

    return k(dst)


def _sc_conv(h, src, dst):
    @functools.partial(
        pl.kernel,
        out_type=jax.ShapeDtypeStruct((NC, NP, D), jnp.float32),
        mesh=_mesh,
        scratch_types=[
            pltpu.VMEM((ZR, D), jnp.float32),
            pltpu.VMEM((CH, D), jnp.float32),
            pltpu.VMEM((CH,), jnp.int32),
            pltpu.VMEM((CH,), jnp.int32),
            pltpu.VMEM_SHARED((NP, D), jnp.float32),
            pltpu.SemaphoreType.DMA,
        ],
        compiler_params=_sc_params,
    )
    def k(h_hbm, src_hbm, dst_hbm, out_hbm, zbuf, rows, ib0, ib1, acc, sem):
        c = lax.axis_index("c")
        s = lax.axis_index("s")
        wid = c * NS + s
        zero16 = jnp.zeros((16,), jnp.float32)

        @pl.loop(0, ZR)
        def _(i):
            @pl.loop(0, D, step=16)
            def _(q):
                zbuf[i, pl.ds(q, 16)] = zero16

        for kk in range(RPT // ZR):
            pltpu.sync_copy(zbuf, acc.at[pl.ds(s * RPT + kk * ZR, ZR)])
        plsc.subcore_barrier()

        @pl.loop(0, NCH)
        def _(j):
            base = wid * EPT + j * CH
            pltpu.sync_copy(src_hbm.at[pl.ds(base, CH)], ib0)
            pltpu.sync_copy(dst_hbm.at[pl.ds(base, CH)], ib1)
            pltpu.async_copy(h_hbm.at[ib0], rows, sem).wait()
            pltpu.sync_copy(rows, acc.at[ib1], add=True)

        plsc.subcore_barrier()
        for kk in range(RPT // ZR):
            r0 = s * RPT + kk * ZR
            pltpu.sync_copy(acc.at[pl.ds(r0, ZR)], out_hbm.at[c].at[pl.ds(r0, ZR)])

    return k(h, src, dst)


def _sc_decode(ta, tb, i0, i1):
    @functools.partial(
        pl.kernel,
        out_type=jax.ShapeDtypeStruct((EP2, 16), jnp.float32),
        mesh=_mesh,
        scratch_types=[
            pltpu.VMEM((CH, 16), jnp.float32),
            pltpu.VMEM((CH, 16), jnp.float32),
            pltpu.VMEM((CH,), jnp.int32),
            pltpu.VMEM((CH,), jnp.int32),
            pltpu.SemaphoreType.DMA,
            pltpu.SemaphoreType.DMA,
        ],
        compiler_params=_sc_params,
    )
    def k(ta_hbm, tb_hbm, i0_hbm, i1_hbm, out_hbm, va, vb, ib0, ib1, sem0, sem1):
        c = lax.axis_index("c")
        s = lax.axis_index("s")
        wid = c * NS + s

        @pl.loop(0, EP2 // NW // CH)
        def _(j):
            base = wid * (EP2 // NW) + j * CH
            pltpu.sync_copy(i0_hbm.at[pl.ds(base, CH)], ib0)
            pltpu.sync_copy(i1_hbm.at[pl.ds(base, CH)], ib1)
            cp0 = pltpu.async_copy(ta_hbm.at[ib0], va, sem0)
            cp1 = pltpu.async_copy(tb_hbm.at[ib1], vb, sem1)
            cp0.wait()
            cp1.wait()

            @pl.loop(0, CH)
            def _(i):
                va[i, :] = va[i, :] + vb[i, :]

            pltpu.sync_copy(va, out_hbm.at[pl.ds(base, CH)])

    return k(ta, tb, i0, i1)


_R = 1024


def _tc_prep(degpair, xp, W1):
    def body(dp_ref, x_ref, w_ref, h_ref, rb_ref):
        deg = dp_ref[0][:, 0:1] + dp_ref[1][:, 0:1]
        r = jnp.where(deg > 0, lax.rsqrt(jnp.maximum(deg, 1.0)), 0.0)
        rb = jnp.broadcast_to(r, (_R, D))
        h = jnp.dot(x_ref[...], w_ref[...], preferred_element_type=jnp.float32)
        h_ref[...] = h * rb
        rb_ref[...] = rb

    return pl.pallas_call(
        body,
        grid=(NP // _R,),
        in_specs=[
            pl.BlockSpec((NC, _R, 16), lambda i: (0, i, 0)),
            pl.BlockSpec((_R, D), lambda i: (i, 0)),
            pl.BlockSpec((D, D), lambda i: (0, 0)),
        ],
        out_specs=[
            pl.BlockSpec((_R, D), lambda i: (i, 0)),
            pl.BlockSpec((_R, D), lambda i: (i, 0)),
        ],
        out_shape=[
            jax.ShapeDtypeStruct((NP, D), jnp.float32),
            jax.ShapeDtypeStruct((NP, D), jnp.float32),
        ],
    )(degpair, xp, W1)


def _tc_mid(zpair, rb, W2):
    def body(zp_ref, rb_ref, w_ref, h_ref):
        rbv = rb_ref[...]
        z = jnp.maximum((zp_ref[0] + zp_ref[1]) * rbv, 0.0)
        h_ref[...] = jnp.dot(z, w_ref[...], preferred_element_type=jnp.float32) * rbv

    return pl.pallas_call(
        body,
        grid=(NP // _R,),
        in_specs=[
            pl.BlockSpec((NC, _R, D), lambda i: (0, i, 0)),
            pl.BlockSpec((_R, D), lambda i: (i, 0)),
            pl.BlockSpec((D, D), lambda i: (0, 0)),
        ],
        out_specs=pl.BlockSpec((_R, D), lambda i: (i, 0)),
        out_shape=jax.ShapeDtypeStruct((NP, D), jnp.float32),
    )(zpair, rb, W2)


def _tc_final(zpair, rb, WA, WB):
    def body(zp_ref, rb_ref, wa_ref, wb_ref, ta_ref, tb_ref):
        z = (zp_ref[0] + zp_ref[1]) * rb_ref[...]
        ta_ref[...] = jnp.dot(z, wa_ref[...], preferred_element_type=jnp.float32)
        tb_ref[...] = jnp.dot(z, wb_ref[...], preferred_element_type=jnp.float32)

    return pl.pallas_call(
        body,
        grid=(NP // _R,),
        in_specs=[
            pl.BlockSpec((NC, _R, D), lambda i: (0, i, 0)),
            pl.BlockSpec((_R, D), lambda i: (i, 0)),
            pl.BlockSpec((D, 16), lambda i: (0, 0)),
            pl.BlockSpec((D, 16), lambda i: (0, 0)),
        ],
        out_specs=[
            pl.BlockSpec((_R, 16), lambda i: (i, 0)),
            pl.BlockSpec((_R, 16), lambda i: (i, 0)),
        ],
        out_shape=[
            jax.ShapeDtypeStruct((NP, 16), jnp.float32),
            jax.ShapeDtypeStruct((NP, 16), jnp.float32),
        ],
    )(zpair, rb, WA, WB)


def kernel(x, edge_index, pos_edge_index, neg_edge_index, W1, W2, Wlin):
    src = edge_index[0]
    dst = edge_index[1]
    ei = jnp.concatenate([pos_edge_index, neg_edge_index], axis=-1)
    i0 = ei[0]
    i1 = ei[1]
    xp = jnp.pad(x, ((0, NP - N), (0, 0)))
    WA = jnp.pad(Wlin[:, :D].T, ((0, 0), (0, 14)))
    WB = jnp.pad(Wlin[:, D:].T, ((0, 0), (0, 14)))

    degpair = _sc_deg(dst)
    h1, rb = _tc_prep(degpair, xp, W1)
    z1p = _sc_conv(h1, src, dst)
    h2 = _tc_mid(z1p, rb, W2)
    z2p = _sc_conv(h2, src, dst)
    ta, tb = _tc_final(z2p, rb, WA, WB)
    outp = _sc_decode(ta, tb, i0, i1)
    return outp[:, :2]
